# Initial kernel scaffold; baseline (speedup 1.0000x reference)
import jax
import jax.numpy as jnp
from jax.experimental import pallas as pl


def kernel(x, edge_index, Wl1, Wr1, b1, Wl2, Wr2, b2):
    raise NotImplementedError("write your pallas kernel here")


# speedup vs baseline: 9.8609x; 9.8609x over previous
import functools

import jax
import jax.numpy as jnp
from jax import lax
from jax.experimental import pallas as pl
from jax.experimental.pallas import tpu as pltpu
from jax.experimental.pallas import tpu_sc as plsc

N = 10000
E = 320000
DF = 128
DH = 16
DC = 40
DC_PAD = 48

NC = 2
NS = 16
NW = NC * NS
CHUNK = 128
N_CHUNKS = -(-E // (NW * CHUNK))
E_PAD = NW * N_CHUNKS * CHUNK
NACC = 10240
ROWS_PT = NACC // NS


@functools.cache
def _seg_sum_kernel(d, with_count):
  mesh = plsc.VectorSubcoreMesh(core_axis_name="c", subcore_axis_name="s")

  out_type = [jax.ShapeDtypeStruct((NC, NACC, d), jnp.float32)]
  scratch = [
      pltpu.VMEM((N_CHUNKS, CHUNK), jnp.int32),
      pltpu.VMEM((N_CHUNKS, CHUNK), jnp.int32),
      pltpu.VMEM((CHUNK, d), jnp.float32),
      pltpu.VMEM((ROWS_PT, d), jnp.float32),
      pltpu.VMEM_SHARED((NACC, d), jnp.float32),
      pltpu.SemaphoreType.DMA,
  ]
  if with_count:
    out_type.append(jax.ShapeDtypeStruct((NC, NACC, 16), jnp.float32))
    scratch += [
        pltpu.VMEM((CHUNK, 16), jnp.float32),
        pltpu.VMEM_SHARED((NACC, 16), jnp.float32),
    ]

  def body(table_h, src_h, dst_h, *rest):
    if with_count:
      out_h, cnt_h, src_v, dst_v, rows_v, zbuf, acc, sem, ones_v, accc = rest
    else:
      out_h, src_v, dst_v, rows_v, zbuf, acc, sem = rest
      cnt_h = ones_v = accc = None
    cid = lax.axis_index("c")
    sid = lax.axis_index("s")
    wid = sid * NC + cid

    zero = jnp.zeros((16,), jnp.float32)
    def zrow(i, _):
      for c0 in range(d // 16):
        zbuf[i, pl.ds(c0 * 16, 16)] = zero
      return 0
    lax.fori_loop(0, ROWS_PT, zrow, 0)
    if with_count:
      one = jnp.ones((16,), jnp.float32)
      def orow(i, _):
        ones_v[i, pl.ds(0, 16)] = one
        return 0
      lax.fori_loop(0, CHUNK, orow, 0)

    base = sid * ROWS_PT
    pltpu.sync_copy(zbuf, acc.at[pl.ds(base, ROWS_PT)])
    if with_count:
      pltpu.sync_copy(zbuf, accc.at[pl.ds(base, ROWS_PT)])
    plsc.subcore_barrier()

    pltpu.sync_copy(src_h.at[wid], src_v)
    pltpu.sync_copy(dst_h.at[wid], dst_v)

    def step(j, _):
      pltpu.async_copy(table_h.at[src_v.at[j]], rows_v, sem).wait()
      pltpu.sync_copy(rows_v, acc.at[dst_v.at[j]], add=True)
      if with_count:
        pltpu.sync_copy(ones_v, accc.at[dst_v.at[j]], add=True)
      return 0
    lax.fori_loop(0, N_CHUNKS, step, 0)

    plsc.subcore_barrier()
    pltpu.sync_copy(acc.at[pl.ds(base, ROWS_PT)],
                    out_h.at[cid, pl.ds(base, ROWS_PT)])
    if with_count:
      pltpu.sync_copy(accc.at[pl.ds(base, ROWS_PT)],
                      cnt_h.at[cid, pl.ds(base, ROWS_PT)])

  return pl.kernel(body, out_type=tuple(out_type), mesh=mesh,
                   scratch_types=tuple(scratch),
                   compiler_params=pltpu.CompilerParams(
                       use_tc_tiling_on_sc=False))


BR = 1000


def _tc_a_body(x_ref, wl_ref, wr_ref, b_ref, xl_ref, xr_ref):
  x = x_ref[...]
  xl_ref[...] = jnp.dot(x, wl_ref[...], preferred_element_type=jnp.float32)
  xr_ref[...] = (jnp.dot(x, wr_ref[...], preferred_element_type=jnp.float32)
                 + b_ref[...])


def _tc_b_body(s_ref, c_ref, xr_ref, wl_ref, wr_ref, b_ref,
               hl_ref, hr_ref, cnt_ref):
  cn = c_ref[0] + c_ref[1]
  mean = (s_ref[0] + s_ref[1]) / jnp.maximum(cn, 1.0)
  h = jnp.maximum(mean + xr_ref[...], 0.0)
  hl_ref[...] = jnp.dot(h, wl_ref[...], preferred_element_type=jnp.float32)
  hr_ref[...] = (jnp.dot(h, wr_ref[...], preferred_element_type=jnp.float32)
                 + b_ref[...])
  cnt_ref[...] = cn


def _tc_c_body(s_ref, cnt_ref, hr_ref, out_ref):
  s = s_ref[0][:, :DC] + s_ref[1][:, :DC]
  c = jnp.maximum(cnt_ref[:, 0:1], 1.0)
  logits = s / c + hr_ref[...]
  m = jnp.max(logits, axis=1, keepdims=True)
  lse = jnp.log(jnp.sum(jnp.exp(logits - m), axis=1, keepdims=True)) + m
  out_ref[...] = logits - lse


def _row_spec(dim):
  return pl.BlockSpec((BR, dim), lambda i: (i, 0))


def _acc_spec(dim):
  return pl.BlockSpec((NC, BR, dim), lambda i: (0, i, 0))


def _full_spec(r, c):
  return pl.BlockSpec((r, c), lambda i: (0, 0))


_tc_a = pl.pallas_call(
    _tc_a_body,
    grid=(N // BR,),
    in_specs=[_row_spec(DF), _full_spec(DF, DH), _full_spec(DF, DH),
              _full_spec(1, DH)],
    out_specs=[_row_spec(DH), _row_spec(DH)],
    out_shape=[jax.ShapeDtypeStruct((N, DH), jnp.float32),
               jax.ShapeDtypeStruct((N, DH), jnp.float32)],
)

_tc_b = pl.pallas_call(
    _tc_b_body,
    grid=(N // BR,),
    in_specs=[_acc_spec(DH), _acc_spec(16), _row_spec(DH),
              _full_spec(DH, DC_PAD), _full_spec(DH, DC), _full_spec(1, DC)],
    out_specs=[_row_spec(DC_PAD), _row_spec(DC), _row_spec(16)],
    out_shape=[jax.ShapeDtypeStruct((N, DC_PAD), jnp.float32),
               jax.ShapeDtypeStruct((N, DC), jnp.float32),
               jax.ShapeDtypeStruct((N, 16), jnp.float32)],
)

_tc_c = pl.pallas_call(
    _tc_c_body,
    grid=(N // BR,),
    in_specs=[_acc_spec(DC_PAD), _row_spec(16), _row_spec(DC)],
    out_specs=pl.BlockSpec((BR, DC), lambda i: (i, 0)),
    out_shape=jax.ShapeDtypeStruct((N, DC), jnp.float32),
)


@jax.jit
def kernel(x, edge_index, Wl1, Wr1, b1, Wl2, Wr2, b2):
  src = edge_index[0].astype(jnp.int32)
  dst = edge_index[1].astype(jnp.int32)
  pad = E_PAD - E
  src_p = jnp.concatenate([src, jnp.zeros((pad,), jnp.int32)])
  dst_p = jnp.concatenate([dst, jnp.full((pad,), N, jnp.int32)])
  src_p = src_p.reshape(NW, N_CHUNKS, CHUNK)
  dst_p = dst_p.reshape(NW, N_CHUNKS, CHUNK)

  xl, xr = _tc_a(x, Wl1, Wr1, b1.reshape(1, DH))
  sums1, cnts1 = _seg_sum_kernel(DH, True)(xl, src_p, dst_p)
  hl, hr, cnt = _tc_b(sums1, cnts1, xr,
                      jnp.pad(Wl2, ((0, 0), (0, DC_PAD - DC))),
                      Wr2, b2.reshape(1, DC))
  (sums2,) = _seg_sum_kernel(DC_PAD, False)(hl, src_p, dst_p)
  return _tc_c(sums2, cnt, hr)
